# baseline (speedup 1.0000x reference)
import functools

import jax
import jax.numpy as jnp
import numpy as np
from jax import lax
from jax.experimental import pallas as pl

N = 64
M = 16384
STATE = 8
CTRL = 4
OBS = 16


def _stage_a_body(obs_ref, ctrl_ref, at_ref, bt_ref, ct_ref,
                  spt_ref, noiset_ref, lwp_ref,
                  predt_ref, invw_ref, lwn_ref, est_ref):
    f = pl.program_id(0)
    spt = spt_ref[0]
    predt = jnp.dot(at_ref[:, :], spt, preferred_element_type=jnp.float32)
    ctrl = ctrl_ref[f, :]
    drive = jnp.dot(bt_ref[:, :], ctrl.reshape(CTRL, 1),
                    preferred_element_type=jnp.float32)
    predt = predt + drive + noiset_ref[0]
    pobst = jnp.dot(ct_ref[:, :], predt, preferred_element_type=jnp.float32)
    err = obs_ref[f, :].reshape(OBS, 1) - pobst
    ll = -0.5 * jnp.sum(err * err, axis=0, keepdims=True)
    lw = lwp_ref[0] + ll
    mx = jnp.max(lw)
    lse = jnp.log(jnp.sum(jnp.exp(lw - mx))) + mx
    lwn = lw - lse
    w = jnp.exp(lwn)
    est = jnp.sum(w * predt, axis=1, keepdims=True)
    predt_ref[0] = predt
    invw_ref[0] = jnp.exp(-lwn)
    lwn_ref[0] = lwn
    est_ref[0] = est.reshape(1, STATE)


def _stage_a(states_prev, log_weights_prev, observations, controls, A, B, C, noise):
    grid = (N,)
    out_shapes = (
        jax.ShapeDtypeStruct((N, STATE, M), jnp.float32),
        jax.ShapeDtypeStruct((N, 1, M), jnp.float32),
        jax.ShapeDtypeStruct((N, 1, M), jnp.float32),
        jax.ShapeDtypeStruct((N, 1, STATE), jnp.float32),
    )
    full = lambda shape: pl.BlockSpec(shape, lambda f: tuple(0 for _ in shape))
    predt, invw, lwn, est = pl.pallas_call(
        _stage_a_body,
        grid=grid,
        in_specs=[
            full((N, OBS)),
            full((N, CTRL)),
            full((STATE, STATE)),
            full((STATE, CTRL)),
            full((OBS, STATE)),
            pl.BlockSpec((1, STATE, M), lambda f: (f, 0, 0)),
            pl.BlockSpec((1, STATE, M), lambda f: (f, 0, 0)),
            pl.BlockSpec((1, 1, M), lambda f: (f, 0, 0)),
        ],
        out_specs=(
            pl.BlockSpec((1, STATE, M), lambda f: (f, 0, 0)),
            pl.BlockSpec((1, 1, M), lambda f: (f, 0, 0)),
            pl.BlockSpec((1, 1, M), lambda f: (f, 0, 0)),
            pl.BlockSpec((1, 1, STATE), lambda f: (f, 0, 0)),
        ),
        out_shape=out_shapes,
    )(observations, controls, A.T, B.T, C.T,
      states_prev.transpose(0, 2, 1), noise.transpose(0, 2, 1),
      log_weights_prev.reshape(N, 1, M))
    return (predt.transpose(0, 2, 1), invw,
            lwn.reshape(N, M), est.reshape(N, STATE))


_SB = 64
_CC = 1024
_NSB = M // _SB
_NC = M // _CC
_TINY = float(np.finfo(np.float32).tiny)

_ROTS = ((13, 15, 26, 6), (17, 29, 16, 24))


def _rotl(x, r):
    return lax.shift_left(x, r) | lax.shift_right_logical(x, 32 - r)


def _threefry_bits(hi, lo):
    k0 = jnp.int32(0)
    k1 = jnp.int32(7)
    k2 = jnp.int32(0x1BD11BDD)
    ks = (k1, k2, k0)
    x0 = hi
    x1 = lo + k1
    for i in range(5):
        for r in _ROTS[i % 2]:
            x0 = x0 + x1
            x1 = _rotl(x1, r) ^ x0
        x0 = x0 + ks[i % 3]
        x1 = x1 + ks[(i + 1) % 3] + jnp.int32(i + 1)
    return x0 ^ x1


def _sampler_body(invw_ref, idx_ref):
    f = pl.program_id(0)
    sb = pl.program_id(1)
    s = sb * _SB + lax.broadcasted_iota(jnp.int32, (_SB, 1), 0)
    hi = lax.shift_right_logical(s, 12)
    lo_base = lax.shift_left(s & 0xFFF, 20) | lax.shift_left(f, 14)
    tiny = jnp.float32(_TINY)

    def chunk(c0, carry):
        best_val, best_idx = carry
        cols = c0 * _CC + lax.broadcasted_iota(jnp.int32, (_SB, _CC), 1)
        bits = _threefry_bits(hi, lo_base | cols)
        fbits = lax.shift_right_logical(bits, 9) | jnp.int32(0x3F800000)
        fl = lax.bitcast_convert_type(fbits, jnp.float32) - jnp.float32(1.0)
        u = jnp.maximum(tiny, fl + tiny)
        t = -jnp.log(u)
        invw = invw_ref[0, 0, pl.ds(c0 * _CC, _CC)].reshape(1, _CC)
        r = t * invw
        cmin = jnp.min(r, axis=1, keepdims=True)
        cidx = jnp.min(jnp.where(r <= cmin, cols, jnp.int32(M)),
                       axis=1, keepdims=True)
        upd = cmin < best_val
        return (jnp.where(upd, cmin, best_val),
                jnp.where(upd, cidx, best_idx))

    init = (jnp.full((_SB, 1), jnp.inf, jnp.float32),
            jnp.zeros((_SB, 1), jnp.int32))
    _, best_idx = lax.fori_loop(0, _NC, chunk, init)
    idx_ref[0, 0] = best_idx


def _sampler(invw3):
    return pl.pallas_call(
        _sampler_body,
        grid=(N, _NSB),
        in_specs=[pl.BlockSpec((1, 1, M), lambda f, sb: (f, 0, 0))],
        out_specs=pl.BlockSpec((1, 1, _SB, 1), lambda f, sb: (f, sb, 0, 0)),
        out_shape=jax.ShapeDtypeStruct((N, _NSB, _SB, 1), jnp.int32),
    )(invw3).reshape(N, M)


def kernel(states_prev, log_weights_prev, observations, controls, A, B, C):
    n, m, state_dim = states_prev.shape
    noise = jax.random.normal(jax.random.key(42), (n, m, state_dim),
                              dtype=jnp.float32) * 0.01
    states_pred, invw3, lwn, state_estimates = _stage_a(
        states_prev, log_weights_prev, observations, controls, A, B, C, noise)
    state_indices = _sampler(invw3)
    states = jnp.take_along_axis(states_pred, state_indices[:, :, None], axis=1)
    log_weights = jnp.full((n, m), -float(np.log(m)), dtype=jnp.float32)
    return (state_estimates, states, log_weights)
